# Initial kernel scaffold; baseline (speedup 1.0000x reference)
import functools

import jax
import jax.numpy as jnp
from jax import lax
from jax.experimental import pallas as pl
from jax.experimental.pallas import tpu as pltpu
from jax.experimental.pallas import tpu_sc as plsc

N = 10000
D = 128
E = 320000
NC = 2
NS = 16
NW = NC * NS
CHUNK = 128
NCHUNK = 2560
CPW = NCHUNK // NW
N_ACC = 10240
DUMP = N
STRIPE = N // NS
SUB = 125
DW = 16

_sc_mesh = plsc.VectorSubcoreMesh(core_axis_name="c", subcore_axis_name="s")


@functools.partial(
    pl.kernel,
    out_type=jax.ShapeDtypeStruct((NC, N, DW), jnp.float32),
    mesh=_sc_mesh,
    scratch_types=[
        pltpu.VMEM((CPW, CHUNK), jnp.int32),
        pltpu.VMEM((CHUNK, DW), jnp.float32),
        pltpu.VMEM_SHARED((N_ACC, DW), jnp.float32),
    ],
)
def _sc_degree(dsts_hbm, ones_hbm, zeros_hbm, out_hbm, dst_v, bnc_v, acc_s):
    c = lax.axis_index("c")
    s = lax.axis_index("s")
    wid = c * NS + s
    pltpu.sync_copy(dsts_hbm.at[pl.ds(wid * CPW, CPW)], dst_v)
    pltpu.sync_copy(zeros_hbm, bnc_v)
    for k in range(STRIPE // SUB):
        pltpu.sync_copy(bnc_v.at[pl.ds(0, SUB)],
                        acc_s.at[pl.ds(s * STRIPE + k * SUB, SUB)])
    plsc.subcore_barrier()
    pltpu.sync_copy(ones_hbm, bnc_v)

    def body(i, carry):
        pltpu.sync_copy(bnc_v, acc_s.at[dst_v.at[i]], add=True)
        return carry

    lax.fori_loop(0, CPW, body, 0)
    plsc.subcore_barrier()
    for k in range(STRIPE // SUB):
        off = s * STRIPE + k * SUB
        pltpu.sync_copy(acc_s.at[pl.ds(off, SUB)], bnc_v.at[pl.ds(0, SUB)])
        pltpu.sync_copy(bnc_v.at[pl.ds(0, SUB)], out_hbm.at[c, pl.ds(off, SUB)])


@functools.partial(
    pl.kernel,
    out_type=jax.ShapeDtypeStruct((NC, N, D), jnp.float32),
    mesh=_sc_mesh,
    scratch_types=[
        pltpu.VMEM((CPW, CHUNK), jnp.int32),
        pltpu.VMEM((CPW, CHUNK), jnp.int32),
        pltpu.VMEM((CHUNK, D), jnp.float32),
        pltpu.VMEM((SUB, D), jnp.float32),
        pltpu.VMEM_SHARED((N_ACC, D), jnp.float32),
        pltpu.SemaphoreType.DMA,
    ],
)
def _sc_edge_agg(h_hbm, srcs_hbm, dsts_hbm, zeros_hbm, out_hbm,
                 src_v, dst_v, rows_v, bnc_v, acc_s, sem):
    c = lax.axis_index("c")
    s = lax.axis_index("s")
    wid = c * NS + s
    pltpu.sync_copy(srcs_hbm.at[pl.ds(wid * CPW, CPW)], src_v)
    pltpu.sync_copy(dsts_hbm.at[pl.ds(wid * CPW, CPW)], dst_v)
    pltpu.sync_copy(zeros_hbm, bnc_v)
    for k in range(STRIPE // SUB):
        pltpu.sync_copy(bnc_v, acc_s.at[pl.ds(s * STRIPE + k * SUB, SUB)])
    plsc.subcore_barrier()

    def body(i, carry):
        pltpu.async_copy(h_hbm.at[src_v.at[i]], rows_v, sem).wait()
        pltpu.sync_copy(rows_v, acc_s.at[dst_v.at[i]], add=True)
        return carry

    lax.fori_loop(0, CPW, body, 0)
    plsc.subcore_barrier()
    for k in range(STRIPE // SUB):
        off = s * STRIPE + k * SUB
        pltpu.sync_copy(acc_s.at[pl.ds(off, SUB)], bnc_v)
        pltpu.sync_copy(bnc_v, out_hbm.at[c, pl.ds(off, SUB)])


RB = 1000
NRB = N // RB


def _tc_embed_body(u_ref, p_ref, wu_ref, bu_ref, wp_ref, bp_ref, w1_ref,
                   degp_ref, h_ref, dinv_ref):
    g = pl.program_id(0)
    deg = degp_ref[0, :, :1] + degp_ref[1, :, :1] + 1.0
    dinv = lax.rsqrt(deg)
    tu = jnp.dot(u_ref[...], wu_ref[...],
                 preferred_element_type=jnp.float32) + bu_ref[...]
    tp = jnp.dot(p_ref[...], wp_ref[...],
                 preferred_element_type=jnp.float32) + bp_ref[...]
    t = jnp.where(g == 0, tu, tp)
    h = jnp.dot(t, w1_ref[...], preferred_element_type=jnp.float32)
    h_ref[...] = h * dinv
    dinv_ref[...] = dinv


_tc_embed = pl.pallas_call(
    _tc_embed_body,
    grid=(2, NRB // 2),
    in_specs=[
        pl.BlockSpec((RB, D), lambda g, j: (j, 0)),
        pl.BlockSpec((RB, D), lambda g, j: (j, 0)),
        pl.BlockSpec((D, D), lambda g, j: (0, 0)),
        pl.BlockSpec((1, D), lambda g, j: (0, 0)),
        pl.BlockSpec((D, D), lambda g, j: (0, 0)),
        pl.BlockSpec((1, D), lambda g, j: (0, 0)),
        pl.BlockSpec((D, D), lambda g, j: (0, 0)),
        pl.BlockSpec((NC, RB, DW), lambda g, j: (0, g * (NRB // 2) + j, 0)),
    ],
    out_specs=[
        pl.BlockSpec((RB, D), lambda g, j: (g * (NRB // 2) + j, 0)),
        pl.BlockSpec((RB, 1), lambda g, j: (g * (NRB // 2) + j, 0)),
    ],
    out_shape=[
        jax.ShapeDtypeStruct((N, D), jnp.float32),
        jax.ShapeDtypeStruct((N, 1), jnp.float32),
    ],
)


def _tc_layer_body(p0_ref, p1_ref, h_ref, dinv_ref, b_ref, w_ref, o_ref):
    dinv = dinv_ref[...]
    comb = (p0_ref[...] + p1_ref[...] + h_ref[...]) * dinv + b_ref[...]
    x = jnp.maximum(comb, 0.0)
    o_ref[...] = jnp.dot(x, w_ref[...],
                         preferred_element_type=jnp.float32) * dinv


def _tc_final_body(p0_ref, p1_ref, h_ref, dinv_ref, b_ref, w_ref, bo_ref,
                   o_ref):
    comb = (p0_ref[...] + p1_ref[...] + h_ref[...]) * dinv_ref[...] + b_ref[...]
    x = jnp.maximum(comb, 0.0)
    o_ref[...] = jnp.dot(x, w_ref[...],
                         preferred_element_type=jnp.float32) + bo_ref[...]


_ROW_SPECS = [
    pl.BlockSpec((RB, D), lambda j: (j, 0)),
    pl.BlockSpec((RB, D), lambda j: (j, 0)),
    pl.BlockSpec((RB, D), lambda j: (j, 0)),
    pl.BlockSpec((RB, 1), lambda j: (j, 0)),
    pl.BlockSpec((1, D), lambda j: (0, 0)),
    pl.BlockSpec((D, D), lambda j: (0, 0)),
]

_tc_layer = pl.pallas_call(
    _tc_layer_body,
    grid=(NRB,),
    in_specs=_ROW_SPECS,
    out_specs=pl.BlockSpec((RB, D), lambda j: (j, 0)),
    out_shape=jax.ShapeDtypeStruct((N, D), jnp.float32),
)

_tc_final = pl.pallas_call(
    _tc_final_body,
    grid=(NRB,),
    in_specs=_ROW_SPECS + [pl.BlockSpec((1, D), lambda j: (0, 0))],
    out_specs=pl.BlockSpec((RB, D), lambda j: (j, 0)),
    out_shape=jax.ShapeDtypeStruct((N, D), jnp.float32),
)


def kernel(user_feats, prod_feats, edge_index, Wu, bu, Wp, bp,
           W1, b1, W2, b2, W3, b3, Wo, bo):
    npad = NCHUNK * CHUNK - E
    srcs = jnp.concatenate(
        [edge_index[0], jnp.zeros((npad,), jnp.int32)]).reshape(NCHUNK, CHUNK)
    dsts = jnp.concatenate(
        [edge_index[1], jnp.full((npad,), DUMP, jnp.int32)]
    ).reshape(NCHUNK, CHUNK)
    ones16 = jnp.ones((CHUNK, DW), jnp.float32)
    zeros16 = jnp.zeros((CHUNK, DW), jnp.float32)
    zerosD = jnp.zeros((SUB, D), jnp.float32)

    degp = _sc_degree(dsts, ones16, zeros16)
    h, dinv = _tc_embed(user_feats, prod_feats, Wu, bu.reshape(1, D), Wp,
                        bp.reshape(1, D), W1, degp)
    for b_cur, w_next in ((b1, W2), (b2, W3)):
        part = _sc_edge_agg(h, srcs, dsts, zerosD)
        h = _tc_layer(part[0], part[1], h, dinv, b_cur.reshape(1, D), w_next)
    part = _sc_edge_agg(h, srcs, dsts, zerosD)
    return _tc_final(part[0], part[1], h, dinv, b3.reshape(1, D), Wo,
                     bo.reshape(1, D))


# speedup vs baseline: 7.1644x; 7.1644x over previous
import functools

import jax
import jax.numpy as jnp
from jax import lax
from jax.experimental import pallas as pl
from jax.experimental.pallas import tpu as pltpu
from jax.experimental.pallas import tpu_sc as plsc

N = 10000
D = 128
E = 320000
NC = 2
NS = 16
NW = NC * NS
CHUNK = 128
NCHUNK = 2560
CPW = NCHUNK // NW
N_ACC = 10240
DUMP = N
STRIPE = N_ACC // NS
SUB = 128
DW = 16

_sc_mesh = plsc.VectorSubcoreMesh(core_axis_name="c", subcore_axis_name="s")


@functools.partial(
    pl.kernel,
    out_type=jax.ShapeDtypeStruct((NC, N_ACC, D), jnp.float32),
    mesh=_sc_mesh,
    scratch_types=[
        pltpu.VMEM((CPW, CHUNK), jnp.int32),
        pltpu.VMEM((CHUNK, D), jnp.float32),
        pltpu.VMEM_SHARED((N_ACC, D), jnp.float32),
    ],
)
def _sc_degree(dsts_hbm, ones_hbm, zeros_hbm, out_hbm, dst_v, bnc_v, acc_s):
    c = lax.axis_index("c")
    s = lax.axis_index("s")
    wid = c * NS + s
    pltpu.sync_copy(dsts_hbm.at[pl.ds(wid * CPW, CPW)], dst_v)
    pltpu.sync_copy(zeros_hbm, bnc_v)
    for k in range(STRIPE // SUB):
        pltpu.sync_copy(bnc_v, acc_s.at[pl.ds(s * STRIPE + k * SUB, SUB)])
    plsc.subcore_barrier()
    pltpu.sync_copy(ones_hbm, bnc_v)

    def body(i, carry):
        pltpu.sync_copy(bnc_v, acc_s.at[dst_v.at[i]], add=True)
        return carry

    lax.fori_loop(0, CPW, body, 0)
    plsc.subcore_barrier()
    for k in range(STRIPE // SUB):
        off = s * STRIPE + k * SUB
        pltpu.sync_copy(acc_s.at[pl.ds(off, SUB)], bnc_v)
        pltpu.sync_copy(bnc_v, out_hbm.at[c, pl.ds(off, SUB)])


@functools.partial(
    pl.kernel,
    out_type=jax.ShapeDtypeStruct((NC, N_ACC, D), jnp.float32),
    mesh=_sc_mesh,
    scratch_types=[
        pltpu.VMEM((CPW, CHUNK), jnp.int32),
        pltpu.VMEM((CPW, CHUNK), jnp.int32),
        pltpu.VMEM((CHUNK, D), jnp.float32),
        pltpu.VMEM_SHARED((N_ACC, D), jnp.float32),
        pltpu.SemaphoreType.DMA,
    ],
)
def _sc_edge_agg(h_hbm, srcs_hbm, dsts_hbm, zeros_hbm, out_hbm,
                 src_v, dst_v, rows_v, acc_s, sem):
    c = lax.axis_index("c")
    s = lax.axis_index("s")
    wid = c * NS + s
    pltpu.sync_copy(srcs_hbm.at[pl.ds(wid * CPW, CPW)], src_v)
    pltpu.sync_copy(dsts_hbm.at[pl.ds(wid * CPW, CPW)], dst_v)
    pltpu.sync_copy(zeros_hbm, rows_v)
    for k in range(STRIPE // SUB):
        pltpu.sync_copy(rows_v, acc_s.at[pl.ds(s * STRIPE + k * SUB, SUB)])
    plsc.subcore_barrier()

    def body(i, carry):
        pltpu.async_copy(h_hbm.at[src_v.at[i]], rows_v, sem).wait()
        pltpu.sync_copy(rows_v, acc_s.at[dst_v.at[i]], add=True)
        return carry

    lax.fori_loop(0, CPW, body, 0)
    plsc.subcore_barrier()
    for k in range(STRIPE // SUB):
        off = s * STRIPE + k * SUB
        pltpu.sync_copy(acc_s.at[pl.ds(off, SUB)], rows_v)
        pltpu.sync_copy(rows_v, out_hbm.at[c, pl.ds(off, SUB)])


RB = 1000
NRB = N // RB


def _tc_embed_body(u_ref, p_ref, wu_ref, bu_ref, wp_ref, bp_ref, w1_ref,
                   degp_ref, h_ref, dinv_ref):
    g = pl.program_id(0)
    deg = degp_ref[0, :, :1] + degp_ref[1, :, :1] + 1.0
    dinv = lax.rsqrt(deg)
    tu = jnp.dot(u_ref[...], wu_ref[...],
                 preferred_element_type=jnp.float32) + bu_ref[...]
    tp = jnp.dot(p_ref[...], wp_ref[...],
                 preferred_element_type=jnp.float32) + bp_ref[...]
    t = jnp.where(g == 0, tu, tp)
    h = jnp.dot(t, w1_ref[...], preferred_element_type=jnp.float32)
    h_ref[...] = h * dinv
    dinv_ref[...] = dinv


_tc_embed = pl.pallas_call(
    _tc_embed_body,
    grid=(2, NRB // 2),
    in_specs=[
        pl.BlockSpec((RB, D), lambda g, j: (j, 0)),
        pl.BlockSpec((RB, D), lambda g, j: (j, 0)),
        pl.BlockSpec((D, D), lambda g, j: (0, 0)),
        pl.BlockSpec((1, D), lambda g, j: (0, 0)),
        pl.BlockSpec((D, D), lambda g, j: (0, 0)),
        pl.BlockSpec((1, D), lambda g, j: (0, 0)),
        pl.BlockSpec((D, D), lambda g, j: (0, 0)),
        pl.BlockSpec((NC, RB, D), lambda g, j: (0, g * (NRB // 2) + j, 0)),
    ],
    out_specs=[
        pl.BlockSpec((RB, D), lambda g, j: (g * (NRB // 2) + j, 0)),
        pl.BlockSpec((RB, 1), lambda g, j: (g * (NRB // 2) + j, 0)),
    ],
    out_shape=[
        jax.ShapeDtypeStruct((N, D), jnp.float32),
        jax.ShapeDtypeStruct((N, 1), jnp.float32),
    ],
)


def _tc_layer_body(p0_ref, p1_ref, h_ref, dinv_ref, b_ref, w_ref, o_ref):
    dinv = dinv_ref[...]
    comb = (p0_ref[...] + p1_ref[...] + h_ref[...]) * dinv + b_ref[...]
    x = jnp.maximum(comb, 0.0)
    o_ref[...] = jnp.dot(x, w_ref[...],
                         preferred_element_type=jnp.float32) * dinv


def _tc_final_body(p0_ref, p1_ref, h_ref, dinv_ref, b_ref, w_ref, bo_ref,
                   o_ref):
    comb = (p0_ref[...] + p1_ref[...] + h_ref[...]) * dinv_ref[...] + b_ref[...]
    x = jnp.maximum(comb, 0.0)
    o_ref[...] = jnp.dot(x, w_ref[...],
                         preferred_element_type=jnp.float32) + bo_ref[...]


_ROW_SPECS = [
    pl.BlockSpec((RB, D), lambda j: (j, 0)),
    pl.BlockSpec((RB, D), lambda j: (j, 0)),
    pl.BlockSpec((RB, D), lambda j: (j, 0)),
    pl.BlockSpec((RB, 1), lambda j: (j, 0)),
    pl.BlockSpec((1, D), lambda j: (0, 0)),
    pl.BlockSpec((D, D), lambda j: (0, 0)),
]

_tc_layer = pl.pallas_call(
    _tc_layer_body,
    grid=(NRB,),
    in_specs=_ROW_SPECS,
    out_specs=pl.BlockSpec((RB, D), lambda j: (j, 0)),
    out_shape=jax.ShapeDtypeStruct((N, D), jnp.float32),
)

_tc_final = pl.pallas_call(
    _tc_final_body,
    grid=(NRB,),
    in_specs=_ROW_SPECS + [pl.BlockSpec((1, D), lambda j: (0, 0))],
    out_specs=pl.BlockSpec((RB, D), lambda j: (j, 0)),
    out_shape=jax.ShapeDtypeStruct((N, D), jnp.float32),
)


def kernel(user_feats, prod_feats, edge_index, Wu, bu, Wp, bp,
           W1, b1, W2, b2, W3, b3, Wo, bo):
    npad = NCHUNK * CHUNK - E
    srcs = jnp.concatenate(
        [edge_index[0], jnp.zeros((npad,), jnp.int32)]).reshape(NCHUNK, CHUNK)
    dsts = jnp.concatenate(
        [edge_index[1], jnp.full((npad,), DUMP, jnp.int32)]
    ).reshape(NCHUNK, CHUNK)
    onesD = jnp.ones((CHUNK, D), jnp.float32)
    zerosD = jnp.zeros((SUB, D), jnp.float32)

    degp = _sc_degree(dsts, onesD, zerosD)
    h, dinv = _tc_embed(user_feats, prod_feats, Wu, bu.reshape(1, D), Wp,
                        bp.reshape(1, D), W1, degp)
    for b_cur, w_next in ((b1, W2), (b2, W3)):
        part = _sc_edge_agg(h, srcs, dsts, zerosD)
        h = _tc_layer(part[0], part[1], h, dinv, b_cur.reshape(1, D), w_next)
    part = _sc_edge_agg(h, srcs, dsts, zerosD)
    return _tc_final(part[0], part[1], h, dinv, b3.reshape(1, D), Wo,
                     bo.reshape(1, D))


# speedup vs baseline: 8.1190x; 1.1332x over previous
import functools

import jax
import jax.numpy as jnp
from jax import lax
from jax.experimental import pallas as pl
from jax.experimental.pallas import tpu as pltpu
from jax.experimental.pallas import tpu_sc as plsc

N = 10000
D = 128
E = 320000
NC = 2
NS = 16
NW = NC * NS
CHUNK = 128
NCHUNK = 2560
CPW = NCHUNK // NW
N_ACC = 10240
DUMP = N
STRIPE = N_ACC // NS
SUB = 128
HCPW = CPW // 2

_sc_mesh = plsc.VectorSubcoreMesh(core_axis_name="c", subcore_axis_name="s")


@functools.partial(
    pl.kernel,
    out_type=jax.ShapeDtypeStruct((NC, N_ACC, D), jnp.float32),
    mesh=_sc_mesh,
    scratch_types=[
        pltpu.VMEM((CPW, CHUNK), jnp.int32),
        pltpu.VMEM((CHUNK, D), jnp.float32),
        pltpu.VMEM_SHARED((N_ACC, D), jnp.float32),
        pltpu.SemaphoreType.DMA,
    ],
)
def _sc_degree(dsts_hbm, ones_hbm, zeros_hbm, out_hbm, dst_v, bnc_v, acc_s,
               sem):
    c = lax.axis_index("c")
    s = lax.axis_index("s")
    wid = c * NS + s
    pltpu.sync_copy(dsts_hbm.at[pl.ds(wid * CPW, CPW)], dst_v)
    pltpu.sync_copy(zeros_hbm, bnc_v)
    for k in range(STRIPE // SUB):
        pltpu.sync_copy(bnc_v, acc_s.at[pl.ds(s * STRIPE + k * SUB, SUB)])
    plsc.subcore_barrier()
    pltpu.sync_copy(ones_hbm, bnc_v)

    def body(g, carry):
        base = g * 8
        for j in range(8):
            pltpu.async_copy(bnc_v, acc_s.at[dst_v.at[base + j]], sem,
                             add=True)
        for j in range(8):
            pltpu.make_async_copy(zeros_hbm, bnc_v, sem).wait()
        return carry

    lax.fori_loop(0, CPW // 8, body, 0)
    plsc.subcore_barrier()
    for k in range(STRIPE // SUB):
        off = s * STRIPE + k * SUB
        pltpu.sync_copy(acc_s.at[pl.ds(off, SUB)], bnc_v)
        pltpu.sync_copy(bnc_v, out_hbm.at[c, pl.ds(off, SUB)])


@functools.partial(
    pl.kernel,
    out_type=jax.ShapeDtypeStruct((NC, N_ACC, D), jnp.float32),
    mesh=_sc_mesh,
    scratch_types=[
        pltpu.VMEM((HCPW, CHUNK), jnp.int32),
        pltpu.VMEM((HCPW, CHUNK), jnp.int32),
        pltpu.VMEM((2, CHUNK, D), jnp.float32),
        pltpu.VMEM_SHARED((N_ACC, D), jnp.float32),
        pltpu.SemaphoreType.DMA,
        pltpu.SemaphoreType.DMA,
    ],
)
def _sc_edge_agg(h_hbm, srcs_hbm, dsts_hbm, zeros_hbm, out_hbm,
                 src_v, dst_v, rows_v, acc_s, sem0, sem1):
    c = lax.axis_index("c")
    s = lax.axis_index("s")
    wid = c * NS + s
    pltpu.sync_copy(zeros_hbm, rows_v.at[0])
    for k in range(STRIPE // SUB):
        pltpu.sync_copy(rows_v.at[0],
                        acc_s.at[pl.ds(s * STRIPE + k * SUB, SUB)])
    plsc.subcore_barrier()

    for half in range(2):
        base = wid * CPW + half * HCPW
        pltpu.sync_copy(srcs_hbm.at[pl.ds(base, HCPW)], src_v)
        pltpu.sync_copy(dsts_hbm.at[pl.ds(base, HCPW)], dst_v)
        pltpu.async_copy(h_hbm.at[src_v.at[0]], rows_v.at[0], sem0)
        pltpu.async_copy(h_hbm.at[src_v.at[1]], rows_v.at[1], sem1)

        def pair(g, carry):
            i = 2 * g
            pltpu.make_async_copy(h_hbm.at[src_v.at[0]], rows_v.at[0],
                                  sem0).wait()
            pltpu.sync_copy(rows_v.at[0], acc_s.at[dst_v.at[i]], add=True)
            pltpu.async_copy(h_hbm.at[src_v.at[i + 2]], rows_v.at[0], sem0)
            pltpu.make_async_copy(h_hbm.at[src_v.at[1]], rows_v.at[1],
                                  sem1).wait()
            pltpu.sync_copy(rows_v.at[1], acc_s.at[dst_v.at[i + 1]], add=True)
            pltpu.async_copy(h_hbm.at[src_v.at[i + 3]], rows_v.at[1], sem1)
            return carry

        lax.fori_loop(0, HCPW // 2 - 1, pair, 0)
        pltpu.make_async_copy(h_hbm.at[src_v.at[0]], rows_v.at[0], sem0).wait()
        pltpu.sync_copy(rows_v.at[0], acc_s.at[dst_v.at[HCPW - 2]], add=True)
        pltpu.make_async_copy(h_hbm.at[src_v.at[1]], rows_v.at[1], sem1).wait()
        pltpu.sync_copy(rows_v.at[1], acc_s.at[dst_v.at[HCPW - 1]], add=True)

    plsc.subcore_barrier()
    for k in range(STRIPE // SUB):
        off = s * STRIPE + k * SUB
        pltpu.sync_copy(acc_s.at[pl.ds(off, SUB)], rows_v.at[0])
        pltpu.sync_copy(rows_v.at[0], out_hbm.at[c, pl.ds(off, SUB)])


RB = 1000
NRB = N // RB


def _tc_embed_body(u_ref, p_ref, wu_ref, bu_ref, wp_ref, bp_ref, w1_ref,
                   degp_ref, h_ref, dinv_ref):
    g = pl.program_id(0)
    deg = degp_ref[0, :, :1] + degp_ref[1, :, :1] + 1.0
    dinv = lax.rsqrt(deg)
    tu = jnp.dot(u_ref[...], wu_ref[...],
                 preferred_element_type=jnp.float32) + bu_ref[...]
    tp = jnp.dot(p_ref[...], wp_ref[...],
                 preferred_element_type=jnp.float32) + bp_ref[...]
    t = jnp.where(g == 0, tu, tp)
    h = jnp.dot(t, w1_ref[...], preferred_element_type=jnp.float32)
    h_ref[...] = h * dinv
    dinv_ref[...] = dinv


_tc_embed = pl.pallas_call(
    _tc_embed_body,
    grid=(2, NRB // 2),
    in_specs=[
        pl.BlockSpec((RB, D), lambda g, j: (j, 0)),
        pl.BlockSpec((RB, D), lambda g, j: (j, 0)),
        pl.BlockSpec((D, D), lambda g, j: (0, 0)),
        pl.BlockSpec((1, D), lambda g, j: (0, 0)),
        pl.BlockSpec((D, D), lambda g, j: (0, 0)),
        pl.BlockSpec((1, D), lambda g, j: (0, 0)),
        pl.BlockSpec((D, D), lambda g, j: (0, 0)),
        pl.BlockSpec((NC, RB, D), lambda g, j: (0, g * (NRB // 2) + j, 0)),
    ],
    out_specs=[
        pl.BlockSpec((RB, D), lambda g, j: (g * (NRB // 2) + j, 0)),
        pl.BlockSpec((RB, 1), lambda g, j: (g * (NRB // 2) + j, 0)),
    ],
    out_shape=[
        jax.ShapeDtypeStruct((N, D), jnp.float32),
        jax.ShapeDtypeStruct((N, 1), jnp.float32),
    ],
)


def _tc_layer_body(p0_ref, p1_ref, h_ref, dinv_ref, b_ref, w_ref, o_ref):
    dinv = dinv_ref[...]
    comb = (p0_ref[...] + p1_ref[...] + h_ref[...]) * dinv + b_ref[...]
    x = jnp.maximum(comb, 0.0)
    o_ref[...] = jnp.dot(x, w_ref[...],
                         preferred_element_type=jnp.float32) * dinv


def _tc_final_body(p0_ref, p1_ref, h_ref, dinv_ref, b_ref, w_ref, bo_ref,
                   o_ref):
    comb = (p0_ref[...] + p1_ref[...] + h_ref[...]) * dinv_ref[...] + b_ref[...]
    x = jnp.maximum(comb, 0.0)
    o_ref[...] = jnp.dot(x, w_ref[...],
                         preferred_element_type=jnp.float32) + bo_ref[...]


_ROW_SPECS = [
    pl.BlockSpec((RB, D), lambda j: (j, 0)),
    pl.BlockSpec((RB, D), lambda j: (j, 0)),
    pl.BlockSpec((RB, D), lambda j: (j, 0)),
    pl.BlockSpec((RB, 1), lambda j: (j, 0)),
    pl.BlockSpec((1, D), lambda j: (0, 0)),
    pl.BlockSpec((D, D), lambda j: (0, 0)),
]

_tc_layer = pl.pallas_call(
    _tc_layer_body,
    grid=(NRB,),
    in_specs=_ROW_SPECS,
    out_specs=pl.BlockSpec((RB, D), lambda j: (j, 0)),
    out_shape=jax.ShapeDtypeStruct((N, D), jnp.float32),
)

_tc_final = pl.pallas_call(
    _tc_final_body,
    grid=(NRB,),
    in_specs=_ROW_SPECS + [pl.BlockSpec((1, D), lambda j: (0, 0))],
    out_specs=pl.BlockSpec((RB, D), lambda j: (j, 0)),
    out_shape=jax.ShapeDtypeStruct((N, D), jnp.float32),
)


def kernel(user_feats, prod_feats, edge_index, Wu, bu, Wp, bp,
           W1, b1, W2, b2, W3, b3, Wo, bo):
    npad = NCHUNK * CHUNK - E
    srcs = jnp.concatenate(
        [edge_index[0], jnp.zeros((npad,), jnp.int32)]).reshape(NCHUNK, CHUNK)
    dsts = jnp.concatenate(
        [edge_index[1], jnp.full((npad,), DUMP, jnp.int32)]
    ).reshape(NCHUNK, CHUNK)
    onesD = jnp.ones((CHUNK, D), jnp.float32)
    zerosD = jnp.zeros((SUB, D), jnp.float32)

    degp = _sc_degree(dsts, onesD, zerosD)
    h, dinv = _tc_embed(user_feats, prod_feats, Wu, bu.reshape(1, D), Wp,
                        bp.reshape(1, D), W1, degp)
    for b_cur, w_next in ((b1, W2), (b2, W3)):
        part = _sc_edge_agg(h, srcs, dsts, zerosD)
        h = _tc_layer(part[0], part[1], h, dinv, b_cur.reshape(1, D), w_next)
    part = _sc_edge_agg(h, srcs, dsts, zerosD)
    return _tc_final(part[0], part[1], h, dinv, b3.reshape(1, D), Wo,
                     bo.reshape(1, D))
